# Initial kernel scaffold; baseline (speedup 1.0000x reference)
import jax
import jax.numpy as jnp
from jax.experimental import pallas as pl


def kernel(input_ids, token_type_ids, word_table, pos_table, type_table, Wt, bt, nn_weight, nn_bias):
    raise NotImplementedError("write your pallas kernel here")


# speedup vs baseline: 4.0699x; 4.0699x over previous
import functools

import jax
import jax.numpy as jnp
from jax import lax
from jax.experimental import pallas as pl
from jax.experimental.pallas import tpu as pltpu
from jax.experimental.pallas import tpu_sc as plsc

VOCAB = 100000
EMB = 128
HID = 512
B = 64
S = 512

NC = 2
NS = 16
NW = NC * NS
TOK = B * S
CHUNK = 128
CHUNKS = TOK // (NW * CHUNK)

NB = 4


def _sc_gather(table, ids3):
  mesh = plsc.VectorSubcoreMesh(core_axis_name="c", subcore_axis_name="s")

  @functools.partial(
      pl.kernel,
      mesh=mesh,
      out_type=jax.ShapeDtypeStruct((TOK, EMB), jnp.float32),
      scratch_types=[
          pltpu.VMEM((CHUNKS, CHUNK), jnp.int32),
          pltpu.VMEM((2, CHUNK, EMB), jnp.float32),
          pltpu.SemaphoreType.DMA,
          pltpu.SemaphoreType.DMA,
          pltpu.SemaphoreType.DMA,
          pltpu.SemaphoreType.DMA,
      ],
  )
  def gather_kernel(table_hbm, ids_hbm, out_hbm, idx_v, rows_v, isem, gsem,
                    wsem0, wsem1):
    wsems = (wsem0, wsem1)
    wid = lax.axis_index("s") * NC + lax.axis_index("c")
    base = wid * (CHUNKS * CHUNK)
    cp = pltpu.make_async_copy(ids_hbm.at[wid], idx_v, isem)
    cp.start()
    cp.wait()

    def gather(j):
      g = pltpu.make_async_copy(table_hbm.at[idx_v.at[j]], rows_v.at[j % 2],
                                gsem)
      g.start()
      return g

    def write(j):
      w = pltpu.make_async_copy(rows_v.at[j % 2],
                                out_hbm.at[pl.ds(base + j * CHUNK, CHUNK)],
                                wsems[j % 2])
      w.start()
      return w

    gathers = [gather(0)]
    writes = []
    for j in range(CHUNKS):
      gathers[j].wait()
      if j + 1 < CHUNKS:
        if j >= 1:
          writes[j - 1].wait()
        gathers.append(gather(j + 1))
      writes.append(write(j))
    writes[CHUNKS - 2].wait()
    writes[CHUNKS - 1].wait()

  return gather_kernel(table, ids3)


def _tc_body(w_ref, tti_ref, wt_ref, bt_ref, pos_ref, type_ref, nw_ref,
             nb_ref, out_ref):
  w = w_ref[...]
  zero = jnp.zeros((NB, 1, EMB), jnp.float32)
  left = jnp.concatenate([w[:, 1:], zero], axis=1)
  right = jnp.concatenate([zero, w[:, :-1]], axis=1)
  cat = jnp.concatenate([left, w, right], axis=2)
  cat2 = cat.reshape(NB * S, 3 * EMB)
  x = jnp.dot(cat2, wt_ref[...], preferred_element_type=jnp.float32)
  x = x.reshape(NB, S, HID)

  tt = type_ref[...]
  tbase = tt[0:1][None]
  tdiff = (tt[1:2] - tt[0:1])[None]
  tti = tti_ref[...]
  emb = x + bt_ref[...][None] + pos_ref[...][None] + tbase + tti * tdiff
  out_ref[...] = emb * nw_ref[...][None] + nb_ref[...][None]


def kernel(input_ids, token_type_ids, word_table, pos_table, type_table, Wt,
           bt, nn_weight, nn_bias):
  ids3 = input_ids.astype(jnp.int32).reshape(NW, CHUNKS, CHUNK)
  w = _sc_gather(word_table, ids3)
  w3 = w.reshape(B, S, EMB)

  wt_t = Wt.T
  tti = token_type_ids.astype(jnp.float32).reshape(B, S, 1)

  grid = (B // NB,)
  out = pl.pallas_call(
      _tc_body,
      grid=grid,
      in_specs=[
          pl.BlockSpec((NB, S, EMB), lambda i: (i, 0, 0)),
          pl.BlockSpec((NB, S, 1), lambda i: (i, 0, 0)),
          pl.BlockSpec((3 * EMB, HID), lambda i: (0, 0)),
          pl.BlockSpec((1, HID), lambda i: (0, 0)),
          pl.BlockSpec((S, HID), lambda i: (0, 0)),
          pl.BlockSpec((2, HID), lambda i: (0, 0)),
          pl.BlockSpec((1, HID), lambda i: (0, 0)),
          pl.BlockSpec((1, HID), lambda i: (0, 0)),
      ],
      out_specs=pl.BlockSpec((NB, S, HID), lambda i: (i, 0, 0)),
      out_shape=jax.ShapeDtypeStruct((B, S, HID), jnp.float32),
  )(w3, tti, wt_t, bt.reshape(1, HID), pos_table, type_table,
    nn_weight.reshape(1, HID), nn_bias.reshape(1, HID))
  return out


# speedup vs baseline: 4.2484x; 1.0439x over previous
import functools

import jax
import jax.numpy as jnp
from jax import lax
from jax.experimental import pallas as pl
from jax.experimental.pallas import tpu as pltpu
from jax.experimental.pallas import tpu_sc as plsc

VOCAB = 100000
EMB = 128
HID = 512
B = 64
S = 512

NC = 2
NS = 16
NW = NC * NS
TOK = B * S
CHUNK = 128
CHUNKS = TOK // (NW * CHUNK)

NB = 8


def _sc_gather(table, ids3):
  mesh = plsc.VectorSubcoreMesh(core_axis_name="c", subcore_axis_name="s")

  @functools.partial(
      pl.kernel,
      mesh=mesh,
      out_type=jax.ShapeDtypeStruct((TOK, EMB), jnp.float32),
      scratch_types=[
          pltpu.VMEM((CHUNKS, CHUNK), jnp.int32),
          pltpu.VMEM((2, CHUNK, EMB), jnp.float32),
          pltpu.SemaphoreType.DMA,
          pltpu.SemaphoreType.DMA,
          pltpu.SemaphoreType.DMA,
          pltpu.SemaphoreType.DMA,
      ],
  )
  def gather_kernel(table_hbm, ids_hbm, out_hbm, idx_v, rows_v, isem, gsem,
                    wsem0, wsem1):
    wsems = (wsem0, wsem1)
    wid = lax.axis_index("s") * NC + lax.axis_index("c")
    base = wid * (CHUNKS * CHUNK)
    cp = pltpu.make_async_copy(ids_hbm.at[wid], idx_v, isem)
    cp.start()
    cp.wait()

    def gather(j):
      g = pltpu.make_async_copy(table_hbm.at[idx_v.at[j]], rows_v.at[j % 2],
                                gsem)
      g.start()
      return g

    def write(j):
      w = pltpu.make_async_copy(rows_v.at[j % 2],
                                out_hbm.at[pl.ds(base + j * CHUNK, CHUNK)],
                                wsems[j % 2])
      w.start()
      return w

    gathers = [gather(0)]
    writes = []
    for j in range(CHUNKS):
      gathers[j].wait()
      if j + 1 < CHUNKS:
        if j >= 1:
          writes[j - 1].wait()
        gathers.append(gather(j + 1))
      writes.append(write(j))
    writes[CHUNKS - 2].wait()
    writes[CHUNKS - 1].wait()

  return gather_kernel(table, ids3)


def _tc_body(w_ref, tti_ref, wt_ref, bt_ref, pos_ref, type_ref, nw_ref,
             nb_ref, out_ref):
  w = w_ref[...]
  zero = jnp.zeros((NB, 1, EMB), jnp.float32)
  left = jnp.concatenate([w[:, 1:], zero], axis=1)
  right = jnp.concatenate([zero, w[:, :-1]], axis=1)
  cat = jnp.concatenate([left, w, right], axis=2)
  cat2 = cat.reshape(NB * S, 3 * EMB)
  x = jnp.dot(cat2, wt_ref[...], preferred_element_type=jnp.float32)
  x = x.reshape(NB, S, HID)

  tt = type_ref[...]
  tbase = tt[0:1][None]
  tdiff = (tt[1:2] - tt[0:1])[None]
  tti = tti_ref[...]
  emb = x + bt_ref[...][None] + pos_ref[...][None] + tbase + tti * tdiff
  out_ref[...] = emb * nw_ref[...][None] + nb_ref[...][None]


def kernel(input_ids, token_type_ids, word_table, pos_table, type_table, Wt,
           bt, nn_weight, nn_bias):
  ids3 = input_ids.astype(jnp.int32).reshape(NW, CHUNKS, CHUNK)
  w = _sc_gather(word_table, ids3)
  w3 = w.reshape(B, S, EMB)

  wt_t = Wt.T
  tti = token_type_ids.astype(jnp.float32).reshape(B, S, 1)

  grid = (B // NB,)
  out = pl.pallas_call(
      _tc_body,
      grid=grid,
      in_specs=[
          pl.BlockSpec((NB, S, EMB), lambda i: (i, 0, 0)),
          pl.BlockSpec((NB, S, 1), lambda i: (i, 0, 0)),
          pl.BlockSpec((3 * EMB, HID), lambda i: (0, 0)),
          pl.BlockSpec((1, HID), lambda i: (0, 0)),
          pl.BlockSpec((S, HID), lambda i: (0, 0)),
          pl.BlockSpec((2, HID), lambda i: (0, 0)),
          pl.BlockSpec((1, HID), lambda i: (0, 0)),
          pl.BlockSpec((1, HID), lambda i: (0, 0)),
      ],
      out_specs=pl.BlockSpec((NB, S, HID), lambda i: (i, 0, 0)),
      out_shape=jax.ShapeDtypeStruct((B, S, HID), jnp.float32),
  )(w3, tti, wt_t, bt.reshape(1, HID), pos_table, type_table,
    nn_weight.reshape(1, HID), nn_bias.reshape(1, HID))
  return out
